# baseline (speedup 1.0000x reference)
import jax
import jax.numpy as jnp
from jax import lax
from jax.experimental import pallas as pl
from jax.experimental.pallas import tpu as pltpu, tpu_sc as plsc

B, L, D = 4096, 201, 128
PL = 208
NC, NS, LANES = 2, 16, 16
NW = NC * NS
BPW = B // NW
NBUF = 4
OUTER = BPW // NBUF
REP = 2048
GPB = (L + LANES - 1) // LANES


def _sc_body(ids_hbm, len_hbm, table_hbm, out_hbm, clen_hbm,
             ids_r, rows_v, len_v, clen_v, semi, semg, sems):
    wid = lax.axis_index("s") * NC + lax.axis_index("c")
    b0 = wid * BPW
    iota = lax.iota(jnp.int32, LANES)

    pltpu.sync_copy(len_hbm.at[pl.ds(b0, BPW)], len_v.at[pl.ds(0, BPW)])
    for j in range(BPW // LANES):
        sl = pl.ds(j * LANES, LANES)
        clen_v[sl] = len_v[sl] + 1
    pltpu.sync_copy(clen_v, clen_hbm.at[pl.ds(b0, BPW)])

    def i_start(k, b):
        pltpu.async_copy(ids_hbm.at[pl.ds((b0 + k) * PL, PL)],
                         ids_r.at[pl.ds(b * PL, PL)], semi.at[b])

    def i_wait(b):
        pltpu.make_async_copy(ids_hbm.at[pl.ds(0, PL)],
                              ids_r.at[pl.ds(b * PL, PL)], semi.at[b]).wait()

    def remap(k, b):
        lnk = len_v[pl.ds(k, LANES)][0]
        for g in range(GPB):
            sl = pl.ds(b * PL + g * LANES, LANES)
            pos = iota + g * LANES
            phase = jnp.bitwise_and(pos + k * 31, REP - 1)
            ids = ids_r[sl]
            ids = jnp.where(pos == lnk, 3, ids)
            ids_r[sl] = ids + phase * 4

    def g_start(k, b):
        pltpu.async_copy(table_hbm.at[ids_r.at[pl.ds(b * PL, 128)]],
                         rows_v.at[b, pl.ds(0, 128)], semg.at[b])
        pltpu.async_copy(table_hbm.at[ids_r.at[pl.ds(b * PL + 128, L - 128)]],
                         rows_v.at[b, pl.ds(128, L - 128)], semg.at[b])

    def g_wait(b):
        pltpu.make_async_copy(table_hbm.at[ids_r.at[pl.ds(b * PL, 128)]],
                              rows_v.at[b, pl.ds(0, 128)], semg.at[b]).wait()
        pltpu.make_async_copy(table_hbm.at[ids_r.at[pl.ds(b * PL + 128, L - 128)]],
                              rows_v.at[b, pl.ds(128, L - 128)],
                              semg.at[b]).wait()

    def s_start(k, b):
        pltpu.async_copy(rows_v.at[b, pl.ds(0, L)], out_hbm.at[b0 + k],
                         sems.at[b])

    def s_wait(b):
        pltpu.make_async_copy(rows_v.at[b, pl.ds(0, L)], out_hbm.at[0],
                              sems.at[b]).wait()

    i_start(0, 0)
    i_wait(0)
    remap(0, 0)
    g_start(0, 0)
    i_start(1, 1)

    def outer(o, carry):
        for bb in range(NBUF):
            k = o * NBUF + bb
            b = bb
            g_wait(b)
            s_start(k, b)
            kn = k + 1
            bn = (bb + 1) % NBUF

            @pl.when(kn < BPW)
            def _():
                i_wait(bn)
                remap(kn, bn)

                @pl.when(kn >= NBUF)
                def _():
                    s_wait(bn)
                g_start(kn, bn)

            @pl.when(k + 2 < BPW)
            def _():
                i_start(k + 2, (bb + 2) % NBUF)
        return carry

    lax.fori_loop(0, OUTER, outer, 0)
    for b in range(NBUF):
        s_wait(b)


def kernel(word_ids, lengths, table, eof_embedding):
    table4 = jnp.concatenate([table, eof_embedding], axis=0)
    table_rep = jnp.tile(table4, (REP, 1))
    ids_flat = jnp.pad(word_ids, ((0, 0), (0, PL - L))).reshape(B * PL)

    mesh = plsc.VectorSubcoreMesh(core_axis_name="c", subcore_axis_name="s")
    rep, char_len = pl.kernel(
        _sc_body,
        out_type=(
            jax.ShapeDtypeStruct((B, L, D), jnp.float32),
            jax.ShapeDtypeStruct((B,), jnp.int32),
        ),
        mesh=mesh,
        scratch_types=[
            pltpu.VMEM((NBUF * PL,), jnp.int32),
            pltpu.VMEM((NBUF, PL, D), jnp.float32),
            pltpu.VMEM((BPW + LANES,), jnp.int32),
            pltpu.VMEM((BPW,), jnp.int32),
            pltpu.SemaphoreType.DMA((NBUF,)),
            pltpu.SemaphoreType.DMA((NBUF,)),
            pltpu.SemaphoreType.DMA((NBUF,)),
        ],
    )(ids_flat, lengths, table_rep)

    return rep, char_len


# speedup vs baseline: 1.0433x; 1.0433x over previous
import jax
import jax.numpy as jnp
from jax import lax
from jax.experimental import pallas as pl
from jax.experimental.pallas import tpu as pltpu, tpu_sc as plsc

B, L, D = 4096, 201, 128
PL = 208
NC, NS, LANES = 2, 16, 16
NW = NC * NS
BPW = B // NW
NBUF = 4
OUTER = BPW // NBUF
REP = 2048
GPB = (L + LANES - 1) // LANES


def _sc_body(ids_hbm, len_hbm, table_hbm, out_hbm, clen_hbm,
             ids_r, rows_v, len_v, clen_v, semi, semg, sems):
    wid = lax.axis_index("s") * NC + lax.axis_index("c")
    b0 = wid * BPW
    iota = lax.iota(jnp.int32, LANES)

    pltpu.sync_copy(len_hbm.at[pl.ds(b0, BPW)], len_v.at[pl.ds(0, BPW)])
    for j in range(BPW // LANES):
        sl = pl.ds(j * LANES, LANES)
        clen_v[sl] = len_v[sl] + 1
    pltpu.sync_copy(clen_v, clen_hbm.at[pl.ds(b0, BPW)])

    def i_start(k, b):
        pltpu.async_copy(ids_hbm.at[pl.ds((b0 + k) * PL, PL)],
                         ids_r.at[pl.ds(b * PL, PL)], semi.at[b])

    def i_wait(b):
        pltpu.make_async_copy(ids_hbm.at[pl.ds(0, PL)],
                              ids_r.at[pl.ds(b * PL, PL)], semi.at[b]).wait()

    def remap(k, b):
        lnk = len_v[pl.ds(k, LANES)][0]
        for g in range(GPB):
            sl = pl.ds(b * PL + g * LANES, LANES)
            pos = iota + g * LANES
            phase = jnp.bitwise_and(pos + k * 31, REP - 1)
            ids = ids_r[sl]
            ids = jnp.where(pos == lnk, 3, ids)
            ids_r[sl] = ids + phase * 4

    def g_start(k, b):
        pltpu.async_copy(table_hbm.at[ids_r.at[pl.ds(b * PL, 128)]],
                         rows_v.at[b, pl.ds(0, 128)], semg.at[b])
        pltpu.async_copy(table_hbm.at[ids_r.at[pl.ds(b * PL + 128, L - 128)]],
                         rows_v.at[b, pl.ds(128, L - 128)], semg.at[b])

    def g_wait(b):
        pltpu.make_async_copy(table_hbm.at[ids_r.at[pl.ds(b * PL, 128)]],
                              rows_v.at[b, pl.ds(0, 128)], semg.at[b]).wait()
        pltpu.make_async_copy(table_hbm.at[ids_r.at[pl.ds(b * PL + 128, L - 128)]],
                              rows_v.at[b, pl.ds(128, L - 128)],
                              semg.at[b]).wait()

    def s_start(k, b):
        pltpu.async_copy(rows_v.at[b, pl.ds(0, L)], out_hbm.at[b0 + k],
                         sems.at[b])

    def s_wait(b):
        pltpu.make_async_copy(rows_v.at[b, pl.ds(0, L)], out_hbm.at[0],
                              sems.at[b]).wait()

    for b in range(NBUF):
        i_start(b, b)
    for b in range(2):
        i_wait(b)
        remap(b, b)
        g_start(b, b)

    def outer(o, carry):
        for bb in range(NBUF):
            k = o * NBUF + bb
            b = bb
            g_wait(b)
            s_start(k, b)
            kn = k + 2
            bn = (bb + 2) % NBUF

            @pl.when(kn < BPW)
            def _():
                i_wait(bn)
                remap(kn, bn)

                @pl.when(kn >= NBUF)
                def _():
                    s_wait(bn)
                g_start(kn, bn)

            @pl.when(k + NBUF < BPW)
            def _():
                i_start(k + NBUF, bb)
        return carry

    lax.fori_loop(0, OUTER, outer, 0)
    for b in range(NBUF):
        s_wait(b)


def kernel(word_ids, lengths, table, eof_embedding):
    table4 = jnp.concatenate([table, eof_embedding], axis=0)
    table_rep = jnp.tile(table4, (REP, 1))
    ids_flat = jnp.pad(word_ids, ((0, 0), (0, PL - L))).reshape(B * PL)

    mesh = plsc.VectorSubcoreMesh(core_axis_name="c", subcore_axis_name="s")
    rep, char_len = pl.kernel(
        _sc_body,
        out_type=(
            jax.ShapeDtypeStruct((B, L, D), jnp.float32),
            jax.ShapeDtypeStruct((B,), jnp.int32),
        ),
        mesh=mesh,
        scratch_types=[
            pltpu.VMEM((NBUF * PL,), jnp.int32),
            pltpu.VMEM((NBUF, PL, D), jnp.float32),
            pltpu.VMEM((BPW + LANES,), jnp.int32),
            pltpu.VMEM((BPW,), jnp.int32),
            pltpu.SemaphoreType.DMA((NBUF,)),
            pltpu.SemaphoreType.DMA((NBUF,)),
            pltpu.SemaphoreType.DMA((NBUF,)),
        ],
    )(ids_flat, lengths, table_rep)

    return rep, char_len
